# baseline (speedup 1.0000x reference)
import dataclasses
import functools

import jax
import jax.numpy as jnp
from jax import lax
from jax.experimental import pallas as pl
from jax.experimental.pallas import tpu as pltpu
from jax.experimental.pallas import tpu_sc as plsc

NC = 2
NS = 16
NW = NC * NS
L = 16

C = 1280
B = 128
FLUSH = 128
FB = FLUSH + B

f32 = jnp.float32
i32 = jnp.int32


def _mesh():
    return plsc.VectorSubcoreMesh(
        core_axis_name="c", subcore_axis_name="s", num_cores=NC, num_subcores=NS
    )


def _cparams():
    cp = pltpu.CompilerParams()
    if "needs_layout_passes" in pltpu.CompilerParams.__dataclass_fields__:
        cp = dataclasses.replace(cp, needs_layout_passes=False)
    if "use_tc_tiling_on_sc" in pltpu.CompilerParams.__dataclass_fields__:
        cp = dataclasses.replace(cp, use_tc_tiling_on_sc=False)
    return cp


def _sig(x):
    return 1.0 / (1.0 + jnp.exp(-x))


def _zero16(ref, n):
    @pl.loop(0, n, step=L)
    def _(i):
        ref[pl.ds(i, L)] = jnp.zeros((L,), ref.dtype)


def _lane_iota():
    return lax.iota(i32, L)


def _al(x):
    return pl.multiple_of(x, 8)


def _forward(mask, W1, W2, b2, src, dst, rel, node_idx, debug=False):
    E = src.shape[0]
    R, EMB, CLASSES = W2.shape
    N = W1.shape[0] // R
    RN = R * N

    assert E % C == 0 and C % B == 0
    NCHUNK = E // C
    ZB = 3360
    rs_tile = ((RN + NS * ZB - 1) // (NS * ZB)) * ZB
    RN_PAD = rs_tile * NS
    nb_tile = ((N + NS * B - 1) // (NS * B)) * B
    N_PAD = nb_tile * NS
    DUMMY = N
    MCAP = ((NCHUNK + NW - 1) // NW) * C + FLUSH

    src = src.astype(i32)
    dst = dst.astype(i32)
    rel = rel.astype(i32)
    nidx16 = jnp.full((L,), node_idx, dtype=i32)

    @functools.partial(
        pl.kernel,
        out_type=[
            jax.ShapeDtypeStruct((NC * RN_PAD,), f32),
            jax.ShapeDtypeStruct((NC * N_PAD,), f32),
            jax.ShapeDtypeStruct((NW * MCAP,), i32),
            jax.ShapeDtypeStruct((NW * MCAP,), i32),
            jax.ShapeDtypeStruct((NW * MCAP,), f32),
            jax.ShapeDtypeStruct((NW * L,), i32),
        ],
        mesh=_mesh(),
        compiler_params=_cparams(),
        scratch_types=[
            pltpu.VMEM_SHARED((RN_PAD,), f32),
            pltpu.VMEM_SHARED((N_PAD,), f32),
            pltpu.VMEM((3360,), f32),
            pltpu.VMEM((C,), i32),
            pltpu.VMEM((C,), i32),
            pltpu.VMEM((C // B, B), i32),
            pltpu.VMEM((B,), f32),
            pltpu.VMEM((FB,), i32),
            pltpu.VMEM((FLUSH,), i32),
            pltpu.VMEM((FLUSH,), i32),
            pltpu.VMEM((FLUSH,), i32),
            pltpu.VMEM((FLUSH,), f32),
            pltpu.VMEM((FLUSH,), i32),
            pltpu.VMEM((L,), i32),
            pltpu.SMEM((2,), i32),
            pltpu.SemaphoreType.DMA,
        ],
    )
    def _k1(src_h, dst_h, rel_h, msk_h, niv_h,
            rs_out_h, nb_out_h, mdst_h, mrel_h, mmsk_h, mcnt_h,
            rs_sh, nb_sh, zb, sbuf, rbuf, vidx2, ones128,
            ecb, e128, d128i, r128i, m128f, d128x, cnt16, pst, sem):
        c = lax.axis_index("c")
        s = lax.axis_index("s")
        w = s * NC + c
        li = _lane_iota()
        ZB = 3360

        _zero16(zb, ZB)

        @pl.loop(0, rs_tile, step=ZB)
        def _(i):
            pltpu.sync_copy(zb, rs_sh.at[pl.ds(_al(s * rs_tile + i), ZB)])

        pltpu.sync_copy(zb.at[pl.ds(0, nb_tile)],
                        nb_sh.at[pl.ds(_al(s * nb_tile), nb_tile)])

        @pl.loop(0, B, step=L)
        def _(i):
            ones128[pl.ds(i, L)] = jnp.ones((L,), f32)

        pltpu.sync_copy(niv_h, cnt16)
        nv = cnt16[...]
        pst[0] = 0
        pst[1] = 0
        plsc.subcore_barrier()

        def flush1(vc):
            ob = pst[1]

            @pl.loop(0, FLUSH // L)
            def _(q):
                e128[pl.ds(q * L, L)] = ecb[pl.ds(q * L, L)]

            pltpu.sync_copy(dst_h.at[e128], d128i)
            pltpu.sync_copy(rel_h.at[e128], r128i)
            pltpu.sync_copy(msk_h.at[e128], m128f)

            @pl.loop(0, FLUSH // L)
            def _(q):
                keep = (li + q * L) < vc
                d128x[pl.ds(q * L, L)] = jnp.where(keep, d128i[pl.ds(q * L, L)],
                                                   DUMMY)

            pltpu.sync_copy(ones128, nb_sh.at[d128x], add=True)
            pltpu.sync_copy(d128x, mdst_h.at[pl.ds(_al(w * MCAP + ob), FLUSH)])
            pltpu.sync_copy(r128i, mrel_h.at[pl.ds(_al(w * MCAP + ob), FLUSH)])
            pltpu.sync_copy(m128f, mmsk_h.at[pl.ds(_al(w * MCAP + ob), FLUSH)])
            pst[1] = ob + FLUSH

            @pl.loop(0, B // L)
            def _(q):
                ecb[pl.ds(q * L, L)] = ecb[pl.ds(FLUSH + q * L, L)]

            pst[0] = pst[0] - FLUSH

        nk = (NCHUNK - 1 - w) // NW + 1

        def chunk_body(k, carry):
            ebase = (w + k * NW) * C
            pltpu.sync_copy(src_h.at[pl.ds(_al(ebase), C)], sbuf)
            pltpu.sync_copy(rel_h.at[pl.ds(_al(ebase), C)], rbuf)

            @pl.loop(0, C // B)
            def _(b):
                base = b * B
                anym = sbuf[pl.ds(base, L)] == nv
                for t in range(1, B // L):
                    anym = anym | (sbuf[pl.ds(base + t * L, L)] == nv)

                @pl.loop(0, B // L)
                def _(t):
                    off = base + t * L
                    vidx2[b, pl.ds(t * L, L)] = (rbuf[pl.ds(off, L)] * N
                                                 + sbuf[pl.ds(off, L)])

                @pl.when(jnp.sum(anym.astype(i32)) > 0)
                def _():
                    for t in range(B // L):
                        off = base + t * L
                        m = sbuf[pl.ds(off, L)] == nv
                        p = pst[0]
                        plsc.store_compressed(ecb.at[pl.ds(p, L)],
                                              li + (ebase + off), mask=m)
                        pst[0] = p + jnp.sum(m.astype(i32))

                    @pl.when(pst[0] >= FLUSH)
                    def _():
                        flush1(FLUSH)

            @pl.loop(0, C // B)
            def _(b):
                pltpu.async_copy(ones128, rs_sh.at[vidx2.at[b]], sem, add=True)

            @pl.loop(0, C // B)
            def _(b):
                pltpu.make_async_copy(ones128, rs_sh.at[vidx2.at[b]], sem).wait()

            return carry

        lax.fori_loop(0, nk, chunk_body, 0)

        pfin = pst[0]

        @pl.when(pfin > 0)
        def _():
            @pl.loop(0, FLUSH // L)
            def _(q):
                keep = (li + q * L) < pfin
                ecb[pl.ds(q * L, L)] = jnp.where(keep, ecb[pl.ds(q * L, L)], 0)

            flush1(pfin)

        cnt16[...] = jnp.full((L,), pst[1], dtype=i32)
        pltpu.sync_copy(cnt16, mcnt_h.at[pl.ds(_al(w * L), L)])

        plsc.subcore_barrier()

        pltpu.sync_copy(rs_sh.at[pl.ds(_al(s * rs_tile), rs_tile)],
                        rs_out_h.at[pl.ds(_al(c * RN_PAD + s * rs_tile), rs_tile)])
        pltpu.sync_copy(nb_sh.at[pl.ds(_al(s * nb_tile), nb_tile)],
                        nb_out_h.at[pl.ds(_al(c * N_PAD + s * nb_tile), nb_tile)])

    rs_all, nb_all, m_dst, m_rel, m_msk, m_cnt = _k1(
        src, dst, rel, mask, nidx16)

    @functools.partial(
        pl.kernel,
        out_type=[
            jax.ShapeDtypeStruct((NC * N_PAD, EMB), f32),
        ],
        mesh=_mesh(),
        compiler_params=_cparams(),
        scratch_types=[
            pltpu.VMEM_SHARED((N_PAD, EMB), f32),
            pltpu.VMEM((N_PAD,), f32),
            pltpu.VMEM((nb_tile,), f32),
            pltpu.VMEM((nb_tile,), f32),
            pltpu.VMEM((200, EMB), f32),
            pltpu.VMEM((C,), i32),
            pltpu.VMEM((FB,), i32),
            pltpu.VMEM((FB,), i32),
            pltpu.VMEM((FLUSH,), i32),
            pltpu.VMEM((FLUSH,), i32),
            pltpu.VMEM((FLUSH,), i32),
            pltpu.VMEM((FLUSH,), i32),
            pltpu.VMEM((FLUSH,), i32),
            pltpu.VMEM((FLUSH,), i32),
            pltpu.VMEM((FLUSH,), f32),
            pltpu.VMEM((FLUSH,), f32),
            pltpu.VMEM((FLUSH,), f32),
            pltpu.VMEM((FLUSH,), f32),
            pltpu.VMEM((FLUSH, EMB), f32),
            pltpu.VMEM((FLUSH // 2, 8 * EMB), f32),
            pltpu.VMEM((FLUSH // 2,), i32),
            pltpu.VMEM((FLUSH,), i32),
            pltpu.SMEM((2,), i32),
        ],
    )
    def _k2(src_h, dst_h, rel_h, msk_h, nb_h, rs_h, w1_h,
            h_out_h,
            h_sh, needed, stg0, stg1, zb16, sbuf,
            ecb, scb, e128, s128, h128, v128, d128i, r128i, m128f,
            c0b, c1b, val128, rows, rows8, h64, sub128, pst):
        c = lax.axis_index("c")
        s = lax.axis_index("s")
        w = s * NC + c
        li = _lane_iota()

        @pl.loop(0, 200)
        def _(i):
            zb16[i] = jnp.zeros((EMB,), f32)

        @pl.loop(0, nb_tile, step=200)
        def _(i):
            pltpu.sync_copy(zb16, h_sh.at[pl.ds(_al(s * nb_tile + i), 200)])

        @pl.loop(0, NS)
        def _(t):
            pltpu.sync_copy(nb_h.at[pl.ds(_al(t * nb_tile), nb_tile)], stg0)
            pltpu.sync_copy(nb_h.at[pl.ds(_al(N_PAD + t * nb_tile), nb_tile)],
                            stg1)

            @pl.loop(0, nb_tile, step=L)
            def _(i):
                needed[pl.ds(_al(t * nb_tile + i), L)] = (
                    stg0[pl.ds(i, L)] + stg1[pl.ds(i, L)])

        pst[0] = 0
        plsc.subcore_barrier()

        def flush2():
            @pl.loop(0, FLUSH // L)
            def _(q):
                e128[pl.ds(q * L, L)] = ecb[pl.ds(q * L, L)]
                s128[pl.ds(q * L, L)] = scb[pl.ds(q * L, L)]

            pltpu.sync_copy(dst_h.at[e128], d128i)
            pltpu.sync_copy(rel_h.at[e128], r128i)
            pltpu.sync_copy(msk_h.at[e128], m128f)

            @pl.loop(0, FLUSH // L)
            def _(q):
                rv = r128i[pl.ds(q * L, L)]
                hor = rv * N + d128i[pl.ds(q * L, L)]
                h128[pl.ds(q * L, L)] = lax.shift_right_logical(hor, 3)
                sub128[pl.ds(q * L, L)] = lax.bitwise_and(hor, 7)
                v128[pl.ds(q * L, L)] = rv * N + s128[pl.ds(q * L, L)]

            pltpu.sync_copy(rs_h.at[v128], c0b)

            @pl.loop(0, FLUSH // L)
            def _(q):
                v128[pl.ds(q * L, L)] = v128[pl.ds(q * L, L)] + RN_PAD

            pltpu.sync_copy(rs_h.at[v128], c1b)

            @pl.loop(0, FLUSH // L)
            def _(q):
                cnt = jnp.maximum(c0b[pl.ds(q * L, L)] + c1b[pl.ds(q * L, L)],
                                  1.0)
                val128[pl.ds(q * L, L)] = _sig(m128f[pl.ds(q * L, L)]) / cnt

            for half in range(2):
                hbase = half * (FLUSH // 2)

                @pl.loop(0, FLUSH // (2 * L))
                def _(q):
                    h64[pl.ds(q * L, L)] = h128[pl.ds(hbase + q * L, L)]

                pltpu.sync_copy(w1_h.at[h64], rows8)

                @pl.loop(0, FLUSH // (2 * L))
                def _(q):
                    vv = val128[pl.ds(hbase + q * L, L)]
                    sb = sub128[pl.ds(hbase + q * L, L)]
                    for kk in range(L):
                        jj = q * L + kk
                        sel = li == kk
                        bc = jnp.sum(jnp.where(sel, vv, 0.0))
                        sj = jnp.max(jnp.where(sel, sb, 0))
                        rows[hbase + jj] = rows8[jj, pl.ds(sj * L, L)] * bc

            pltpu.sync_copy(rows, h_sh.at[s128], add=True)

            @pl.loop(0, B // L)
            def _(q):
                ecb[pl.ds(q * L, L)] = ecb[pl.ds(FLUSH + q * L, L)]
                scb[pl.ds(q * L, L)] = scb[pl.ds(FLUSH + q * L, L)]

            pst[0] = pst[0] - FLUSH

        nk = (NCHUNK - 1 - w) // NW + 1

        def chunk_body(k, carry):
            ebase = (w + k * NW) * C
            pltpu.sync_copy(src_h.at[pl.ds(_al(ebase), C)], sbuf)

            @pl.loop(0, C // B)
            def _(b):
                base = b * B
                anym = plsc.load_gather(needed, [sbuf[pl.ds(base, L)]]) > 0.0
                for t in range(1, B // L):
                    anym = anym | (plsc.load_gather(
                        needed, [sbuf[pl.ds(base + t * L, L)]]) > 0.0)

                @pl.when(jnp.sum(anym.astype(i32)) > 0)
                def _():
                    for t in range(B // L):
                        off = base + t * L
                        sv = sbuf[pl.ds(off, L)]
                        m = plsc.load_gather(needed, [sv]) > 0.0
                        p = pst[0]
                        plsc.store_compressed(ecb.at[pl.ds(p, L)],
                                              li + (ebase + off), mask=m)
                        plsc.store_compressed(scb.at[pl.ds(p, L)], sv, mask=m)
                        pst[0] = p + jnp.sum(m.astype(i32))

                    @pl.when(pst[0] >= FLUSH)
                    def _():
                        flush2()

            return carry

        lax.fori_loop(0, nk, chunk_body, 0)

        pfin = pst[0]

        @pl.when(pfin > 0)
        def _():
            @pl.loop(0, FLUSH // L)
            def _(q):
                idxv = li + q * L
                keep = idxv < pfin
                ecb[pl.ds(q * L, L)] = jnp.where(keep, ecb[pl.ds(q * L, L)], 0)
                scb[pl.ds(q * L, L)] = jnp.where(keep, scb[pl.ds(q * L, L)],
                                                 DUMMY)

            flush2()

        plsc.subcore_barrier()

        @pl.loop(0, nb_tile, step=200)
        def _(i):
            pltpu.sync_copy(h_sh.at[pl.ds(_al(s * nb_tile + i), 200)],
                            h_out_h.at[pl.ds(_al(c * N_PAD + s * nb_tile + i),
                                             200)])

    (h_all,) = _k2(src, dst, rel, mask, nb_all, rs_all,
                   W1.reshape(RN // 8, 8 * EMB))

    @functools.partial(
        pl.kernel,
        out_type=[jax.ShapeDtypeStruct((NC, R, EMB), f32)],
        mesh=_mesh(),
        compiler_params=_cparams(),
        scratch_types=[
            pltpu.VMEM_SHARED((NS, R, EMB), f32),
            pltpu.VMEM((R, EMB), f32),
            pltpu.VMEM((R, EMB), f32),
            pltpu.VMEM((R, EMB), f32),
            pltpu.VMEM((2 * L,), f32),
            pltpu.VMEM((L,), i32),
            pltpu.VMEM((L,), f32),
            pltpu.VMEM((L,), f32),
            pltpu.VMEM((L,), i32),
            pltpu.VMEM((FLUSH,), i32),
            pltpu.VMEM((FLUSH,), i32),
            pltpu.VMEM((FLUSH,), i32),
            pltpu.VMEM((FLUSH,), f32),
            pltpu.VMEM((FLUSH, EMB), f32),
            pltpu.VMEM((FLUSH, EMB), f32),
        ],
    )
    def _k3(mdst_h, mrel_h, mmsk_h, mcnt_h, h_h, rs_h, niv_h,
            hvp_h,
            hv_sh, hv_l, acc, stg, counts, idxb, ca, cbv, cvec,
            d128, d128b, r128, m128, rows0, rows1):
        c = lax.axis_index("c")
        s = lax.axis_index("s")
        w = s * NC + c
        li = _lane_iota()

        pltpu.sync_copy(niv_h, cvec)
        nv = cvec[...]

        idxb[...] = li * N + nv
        pltpu.sync_copy(rs_h.at[idxb], ca)
        idxb[...] = idxb[...] + RN_PAD
        pltpu.sync_copy(rs_h.at[idxb], cbv)
        counts[pl.ds(0, L)] = ca[...] + cbv[...]
        idxb[...] = (li * 0 + L) * N + nv
        pltpu.sync_copy(rs_h.at[idxb], ca)
        idxb[...] = idxb[...] + RN_PAD
        pltpu.sync_copy(rs_h.at[idxb], cbv)
        counts[pl.ds(L, L)] = ca[...] + cbv[...]

        @pl.loop(0, R)
        def _(r):
            hv_l[r] = jnp.zeros((EMB,), f32)

        pltpu.sync_copy(mcnt_h.at[pl.ds(_al(w * L), L)], cvec)
        cw = jnp.max(cvec[...])

        def batch_body(j, carry):
            pltpu.sync_copy(mdst_h.at[pl.ds(_al(w * MCAP + j * FLUSH), FLUSH)],
                            d128)
            pltpu.sync_copy(mrel_h.at[pl.ds(_al(w * MCAP + j * FLUSH), FLUSH)],
                            r128)
            pltpu.sync_copy(mmsk_h.at[pl.ds(_al(w * MCAP + j * FLUSH), FLUSH)],
                            m128)
            pltpu.sync_copy(h_h.at[d128], rows0)

            @pl.loop(0, FLUSH // L)
            def _(q):
                d128b[pl.ds(q * L, L)] = d128[pl.ds(q * L, L)] + N_PAD

            pltpu.sync_copy(h_h.at[d128b], rows1)

            @pl.loop(0, FLUSH // L)
            def _(q):
                r16 = r128[pl.ds(q * L, L)]
                d16 = d128[pl.ds(q * L, L)]
                cntv = plsc.load_gather(counts, [r16])
                vv = jnp.where(
                    d16 == DUMMY, 0.0,
                    _sig(m128[pl.ds(q * L, L)]) / jnp.maximum(cntv, 1.0))
                for kk in range(L):
                    jj = q * L + kk
                    sel = li == kk
                    rel_j = jnp.max(jnp.where(sel, r16, 0))
                    vj = jnp.sum(jnp.where(sel, vv, 0.0))
                    row = jnp.maximum(rows0[jj] + rows1[jj], 0.0)
                    hv_l[rel_j] = hv_l[rel_j] + vj * row

            return carry

        lax.fori_loop(0, cw // FLUSH, batch_body, 0)

        pltpu.sync_copy(hv_l, hv_sh.at[s])
        plsc.subcore_barrier()

        @pl.when(s == 0)
        def _():
            @pl.loop(0, R)
            def _(r):
                acc[r] = jnp.zeros((EMB,), f32)

            @pl.loop(0, NS)
            def _(t):
                pltpu.sync_copy(hv_sh.at[t], stg)

                @pl.loop(0, R)
                def _(r):
                    acc[r] = acc[r] + stg[r]

            pltpu.sync_copy(acc, hvp_h.at[c])

    (hvp,) = _k3(m_dst, m_rel, m_msk, m_cnt, h_all, rs_all, nidx16)

    W2p = jnp.pad(W2.astype(f32), ((0, 0), (0, 0), (0, L - CLASSES)))
    W2p = W2p.reshape(R * EMB, L)
    b2p = jnp.concatenate([b2.astype(f32), jnp.full((L - CLASSES,), -1e30, f32)])

    @functools.partial(
        pl.kernel,
        out_type=[jax.ShapeDtypeStruct((L,), f32)],
        mesh=_mesh(),
        compiler_params=_cparams(),
        scratch_types=[
            pltpu.VMEM((R, EMB), f32),
            pltpu.VMEM((R, EMB), f32),
            pltpu.VMEM((R * EMB, L), f32),
            pltpu.VMEM((L,), f32),
            pltpu.VMEM((L,), f32),
        ],
    )
    def _k4(hvp_h, w2_h, b2_h, res_h, a0, a1, w2v, bb, lgr):
        c = lax.axis_index("c")
        s = lax.axis_index("s")

        @pl.when(jnp.logical_and(c == 0, s == 0))
        def _():
            pltpu.sync_copy(hvp_h.at[0], a0)
            pltpu.sync_copy(hvp_h.at[1], a1)
            pltpu.sync_copy(w2_h, w2v)
            pltpu.sync_copy(b2_h, bb)
            lgr[...] = bb[...]
            li = _lane_iota()

            @pl.loop(0, R)
            def _(r):
                hrow = a0[r] + a1[r]
                for e in range(EMB):
                    bc = jnp.sum(jnp.where(li == e, hrow, 0.0))
                    lgr[...] = lgr[...] + bc * w2v[r * EMB + e]

            lg = lgr[...]
            mx = jnp.max(lg)
            ex = jnp.exp(lg - mx)
            sm = jnp.sum(ex)
            lgr[...] = ex / sm
            pltpu.sync_copy(lgr, res_h)

    (res,) = _k4(hvp, W2p, b2p)
    if debug:
        return dict(rs0=rs_all[:RN_PAD], rs1=rs_all[RN_PAD:],
                    nb0=nb_all[:N_PAD], nb1=nb_all[N_PAD:], m_dst=m_dst,
                    m_rel=m_rel, m_msk=m_msk, m_cnt=m_cnt,
                    h0=h_all[:N_PAD], h1=h_all[N_PAD:],
                    hvp=hvp, res=res)
    return res[:CLASSES]


def kernel(mask, W1, W2, b2, src, dst, rel, node_idx):
    return _forward(mask, W1, W2, b2, src, dst, rel, node_idx)


# speedup vs baseline: 1.1743x; 1.1743x over previous
import dataclasses
import functools

import jax
import jax.numpy as jnp
from jax import lax
from jax.experimental import pallas as pl
from jax.experimental.pallas import tpu as pltpu
from jax.experimental.pallas import tpu_sc as plsc

NC = 2
NS = 16
NW = NC * NS
L = 16

C = 1280
B = 128
FLUSH = 128
FB = FLUSH + B

f32 = jnp.float32
i32 = jnp.int32


def _mesh():
    return plsc.VectorSubcoreMesh(
        core_axis_name="c", subcore_axis_name="s", num_cores=NC, num_subcores=NS
    )


def _cparams():
    cp = pltpu.CompilerParams()
    if "needs_layout_passes" in pltpu.CompilerParams.__dataclass_fields__:
        cp = dataclasses.replace(cp, needs_layout_passes=False)
    if "use_tc_tiling_on_sc" in pltpu.CompilerParams.__dataclass_fields__:
        cp = dataclasses.replace(cp, use_tc_tiling_on_sc=False)
    return cp


def _sig(x):
    return 1.0 / (1.0 + jnp.exp(-x))


def _zero16(ref, n):
    @pl.loop(0, n, step=L)
    def _(i):
        ref[pl.ds(i, L)] = jnp.zeros((L,), ref.dtype)


def _lane_iota():
    return lax.iota(i32, L)


def _al(x):
    return pl.multiple_of(x, 8)


def _forward(mask, W1, W2, b2, src, dst, rel, node_idx):
    E = src.shape[0]
    R, EMB, CLASSES = W2.shape
    N = W1.shape[0] // R
    RN = R * N

    assert E % C == 0 and C % B == 0
    NCHUNK = E // C
    ZB = 3360
    rs_tile = ((RN + NS * ZB - 1) // (NS * ZB)) * ZB
    RN_PAD = rs_tile * NS
    nb_tile = ((N + NS * B - 1) // (NS * B)) * B
    N_PAD = nb_tile * NS
    DUMMY = N
    MCAP = ((NCHUNK + NW - 1) // NW) * C + FLUSH

    src = src.astype(i32)
    dst = dst.astype(i32)
    rel = rel.astype(i32)
    nidx16 = jnp.full((L,), node_idx, dtype=i32)

    @functools.partial(
        pl.kernel,
        out_type=[
            jax.ShapeDtypeStruct((NC * RN_PAD,), f32),
            jax.ShapeDtypeStruct((NC * N_PAD,), f32),
            jax.ShapeDtypeStruct((NW * MCAP,), i32),
            jax.ShapeDtypeStruct((NW * MCAP,), i32),
            jax.ShapeDtypeStruct((NW * MCAP,), f32),
            jax.ShapeDtypeStruct((NW * L,), i32),
        ],
        mesh=_mesh(),
        compiler_params=_cparams(),
        scratch_types=[
            pltpu.VMEM_SHARED((RN_PAD,), f32),
            pltpu.VMEM_SHARED((N_PAD,), f32),
            pltpu.VMEM((3360,), f32),
            pltpu.VMEM((C,), i32),
            pltpu.VMEM((C,), i32),
            pltpu.VMEM((C // B, B), i32),
            pltpu.VMEM((B,), f32),
            pltpu.VMEM((FB,), i32),
            pltpu.VMEM((FLUSH,), i32),
            pltpu.VMEM((FLUSH,), i32),
            pltpu.VMEM((FLUSH,), i32),
            pltpu.VMEM((FLUSH,), f32),
            pltpu.VMEM((FLUSH,), i32),
            pltpu.VMEM((L,), i32),
            pltpu.SMEM((2,), i32),
            pltpu.SemaphoreType.DMA,
        ],
    )
    def _k1(src_h, dst_h, rel_h, msk_h, niv_h,
            rs_out_h, nb_out_h, mdst_h, mrel_h, mmsk_h, mcnt_h,
            rs_sh, nb_sh, zb, sbuf, rbuf, vidx2, ones128,
            ecb, e128, d128i, r128i, m128f, d128x, cnt16, pst, sem):
        c = lax.axis_index("c")
        s = lax.axis_index("s")
        w = s * NC + c
        li = _lane_iota()
        ZB = 3360

        _zero16(zb, ZB)

        @pl.loop(0, rs_tile, step=ZB)
        def _(i):
            pltpu.sync_copy(zb, rs_sh.at[pl.ds(_al(s * rs_tile + i), ZB)])

        pltpu.sync_copy(zb.at[pl.ds(0, nb_tile)],
                        nb_sh.at[pl.ds(_al(s * nb_tile), nb_tile)])

        @pl.loop(0, B, step=L)
        def _(i):
            ones128[pl.ds(i, L)] = jnp.ones((L,), f32)

        pltpu.sync_copy(niv_h, cnt16)
        nv = cnt16[...]
        pst[0] = 0
        pst[1] = 0
        plsc.subcore_barrier()

        def flush1(vc):
            ob = pst[1]

            @pl.loop(0, FLUSH // L)
            def _(q):
                e128[pl.ds(q * L, L)] = ecb[pl.ds(q * L, L)]

            pltpu.sync_copy(dst_h.at[e128], d128i)
            pltpu.sync_copy(rel_h.at[e128], r128i)
            pltpu.sync_copy(msk_h.at[e128], m128f)

            @pl.loop(0, FLUSH // L)
            def _(q):
                keep = (li + q * L) < vc
                d128x[pl.ds(q * L, L)] = jnp.where(keep, d128i[pl.ds(q * L, L)],
                                                   DUMMY)

            pltpu.sync_copy(ones128, nb_sh.at[d128x], add=True)
            pltpu.sync_copy(d128x, mdst_h.at[pl.ds(_al(w * MCAP + ob), FLUSH)])
            pltpu.sync_copy(r128i, mrel_h.at[pl.ds(_al(w * MCAP + ob), FLUSH)])
            pltpu.sync_copy(m128f, mmsk_h.at[pl.ds(_al(w * MCAP + ob), FLUSH)])
            pst[1] = ob + FLUSH

            @pl.loop(0, B // L)
            def _(q):
                ecb[pl.ds(q * L, L)] = ecb[pl.ds(FLUSH + q * L, L)]

            pst[0] = pst[0] - FLUSH

        nk = (NCHUNK - 1 - w) // NW + 1

        def chunk_body(k, carry):
            ebase = (w + k * NW) * C
            pltpu.sync_copy(src_h.at[pl.ds(_al(ebase), C)], sbuf)
            pltpu.sync_copy(rel_h.at[pl.ds(_al(ebase), C)], rbuf)

            @pl.loop(0, C // B)
            def _(b):
                base = b * B
                anym = sbuf[pl.ds(base, L)] == nv
                for t in range(1, B // L):
                    anym = anym | (sbuf[pl.ds(base + t * L, L)] == nv)

                @pl.loop(0, B // L)
                def _(t):
                    off = base + t * L
                    vidx2[b, pl.ds(t * L, L)] = (rbuf[pl.ds(off, L)] * N
                                                 + sbuf[pl.ds(off, L)])

                @pl.when(jnp.sum(anym.astype(i32)) > 0)
                def _():
                    for t in range(B // L):
                        off = base + t * L
                        m = sbuf[pl.ds(off, L)] == nv
                        p = pst[0]
                        plsc.store_compressed(ecb.at[pl.ds(p, L)],
                                              li + (ebase + off), mask=m)
                        pst[0] = p + jnp.sum(m.astype(i32))

                    @pl.when(pst[0] >= FLUSH)
                    def _():
                        flush1(FLUSH)

            @pl.loop(0, C // B)
            def _(b):
                pltpu.async_copy(ones128, rs_sh.at[vidx2.at[b]], sem, add=True)

            @pl.loop(0, C // B)
            def _(b):
                pltpu.make_async_copy(ones128, rs_sh.at[vidx2.at[b]], sem).wait()

            return carry

        lax.fori_loop(0, nk, chunk_body, 0)

        pfin = pst[0]

        @pl.when(pfin > 0)
        def _():
            @pl.loop(0, FLUSH // L)
            def _(q):
                keep = (li + q * L) < pfin
                ecb[pl.ds(q * L, L)] = jnp.where(keep, ecb[pl.ds(q * L, L)], 0)

            flush1(pfin)

        cnt16[...] = jnp.full((L,), pst[1], dtype=i32)
        pltpu.sync_copy(cnt16, mcnt_h.at[pl.ds(_al(w * L), L)])

        plsc.subcore_barrier()

        pltpu.sync_copy(rs_sh.at[pl.ds(_al(s * rs_tile), rs_tile)],
                        rs_out_h.at[pl.ds(_al(c * RN_PAD + s * rs_tile), rs_tile)])
        pltpu.sync_copy(nb_sh.at[pl.ds(_al(s * nb_tile), nb_tile)],
                        nb_out_h.at[pl.ds(_al(c * N_PAD + s * nb_tile), nb_tile)])

    rs_all, nb_all, m_dst, m_rel, m_msk, m_cnt = _k1(
        src, dst, rel, mask, nidx16)

    @functools.partial(
        pl.kernel,
        out_type=[
            jax.ShapeDtypeStruct((NC * N_PAD, EMB), f32),
        ],
        mesh=_mesh(),
        compiler_params=_cparams(),
        scratch_types=[
            pltpu.VMEM_SHARED((N_PAD, EMB), f32),
            pltpu.VMEM((N_PAD,), f32),
            pltpu.VMEM((nb_tile,), f32),
            pltpu.VMEM((nb_tile,), f32),
            pltpu.VMEM((200, EMB), f32),
            pltpu.VMEM((C,), i32),
            pltpu.VMEM((FB,), i32),
            pltpu.VMEM((FB,), i32),
            pltpu.VMEM((FLUSH,), i32),
            pltpu.VMEM((FLUSH,), i32),
            pltpu.VMEM((FLUSH,), i32),
            pltpu.VMEM((FLUSH,), i32),
            pltpu.VMEM((FLUSH,), i32),
            pltpu.VMEM((FLUSH,), i32),
            pltpu.VMEM((FLUSH,), f32),
            pltpu.VMEM((FLUSH,), f32),
            pltpu.VMEM((FLUSH,), f32),
            pltpu.VMEM((FLUSH,), f32),
            pltpu.VMEM((FLUSH, EMB), f32),
            pltpu.SMEM((2,), i32),
        ],
    )
    def _k2(src_h, dst_h, rel_h, msk_h, nb_h, rs_h, w1_h,
            h_out_h,
            h_sh, needed, stg0, stg1, zb16, sbuf,
            ecb, scb, e128, s128, h128, v128, d128i, r128i, m128f,
            c0b, c1b, val128, rows, pst):
        c = lax.axis_index("c")
        s = lax.axis_index("s")
        w = s * NC + c
        li = _lane_iota()

        @pl.loop(0, 200)
        def _(i):
            zb16[i] = jnp.zeros((EMB,), f32)

        @pl.loop(0, nb_tile, step=200)
        def _(i):
            pltpu.sync_copy(zb16, h_sh.at[pl.ds(_al(s * nb_tile + i), 200)])

        @pl.loop(0, NS)
        def _(t):
            pltpu.sync_copy(nb_h.at[pl.ds(_al(t * nb_tile), nb_tile)], stg0)
            pltpu.sync_copy(nb_h.at[pl.ds(_al(N_PAD + t * nb_tile), nb_tile)],
                            stg1)

            @pl.loop(0, nb_tile, step=L)
            def _(i):
                needed[pl.ds(_al(t * nb_tile + i), L)] = (
                    stg0[pl.ds(i, L)] + stg1[pl.ds(i, L)])

        pst[0] = 0
        plsc.subcore_barrier()

        def flush2():
            @pl.loop(0, FLUSH // L)
            def _(q):
                e128[pl.ds(q * L, L)] = ecb[pl.ds(q * L, L)]
                s128[pl.ds(q * L, L)] = scb[pl.ds(q * L, L)]

            pltpu.sync_copy(dst_h.at[e128], d128i)
            pltpu.sync_copy(rel_h.at[e128], r128i)
            pltpu.sync_copy(msk_h.at[e128], m128f)

            @pl.loop(0, FLUSH // L)
            def _(q):
                rv = r128i[pl.ds(q * L, L)]
                h128[pl.ds(q * L, L)] = rv * N + d128i[pl.ds(q * L, L)]
                v128[pl.ds(q * L, L)] = rv * N + s128[pl.ds(q * L, L)]

            pltpu.sync_copy(rs_h.at[v128], c0b)

            @pl.loop(0, FLUSH // L)
            def _(q):
                v128[pl.ds(q * L, L)] = v128[pl.ds(q * L, L)] + RN_PAD

            pltpu.sync_copy(rs_h.at[v128], c1b)

            @pl.loop(0, FLUSH // L)
            def _(q):
                cnt = jnp.maximum(c0b[pl.ds(q * L, L)] + c1b[pl.ds(q * L, L)],
                                  1.0)
                val128[pl.ds(q * L, L)] = _sig(m128f[pl.ds(q * L, L)]) / cnt

            pltpu.sync_copy(w1_h.at[h128], rows)

            @pl.loop(0, FLUSH // L)
            def _(q):
                vv = val128[pl.ds(q * L, L)]
                for kk in range(L):
                    jj = q * L + kk
                    bc = jnp.sum(jnp.where(li == kk, vv, 0.0))
                    rows[jj] = rows[jj] * bc

            pltpu.sync_copy(rows, h_sh.at[s128], add=True)

            @pl.loop(0, B // L)
            def _(q):
                ecb[pl.ds(q * L, L)] = ecb[pl.ds(FLUSH + q * L, L)]
                scb[pl.ds(q * L, L)] = scb[pl.ds(FLUSH + q * L, L)]

            pst[0] = pst[0] - FLUSH

        nk = (NCHUNK - 1 - w) // NW + 1

        def chunk_body(k, carry):
            ebase = (w + k * NW) * C
            pltpu.sync_copy(src_h.at[pl.ds(_al(ebase), C)], sbuf)

            @pl.loop(0, C // B)
            def _(b):
                base = b * B
                anym = plsc.load_gather(needed, [sbuf[pl.ds(base, L)]]) > 0.0
                for t in range(1, B // L):
                    anym = anym | (plsc.load_gather(
                        needed, [sbuf[pl.ds(base + t * L, L)]]) > 0.0)

                @pl.when(jnp.sum(anym.astype(i32)) > 0)
                def _():
                    for t in range(B // L):
                        off = base + t * L
                        sv = sbuf[pl.ds(off, L)]
                        m = plsc.load_gather(needed, [sv]) > 0.0
                        p = pst[0]
                        plsc.store_compressed(ecb.at[pl.ds(p, L)],
                                              li + (ebase + off), mask=m)
                        plsc.store_compressed(scb.at[pl.ds(p, L)], sv, mask=m)
                        pst[0] = p + jnp.sum(m.astype(i32))

                    @pl.when(pst[0] >= FLUSH)
                    def _():
                        flush2()

            return carry

        lax.fori_loop(0, nk, chunk_body, 0)

        pfin = pst[0]

        @pl.when(pfin > 0)
        def _():
            @pl.loop(0, FLUSH // L)
            def _(q):
                idxv = li + q * L
                keep = idxv < pfin
                ecb[pl.ds(q * L, L)] = jnp.where(keep, ecb[pl.ds(q * L, L)], 0)
                scb[pl.ds(q * L, L)] = jnp.where(keep, scb[pl.ds(q * L, L)],
                                                 DUMMY)

            flush2()

        plsc.subcore_barrier()

        @pl.loop(0, nb_tile, step=200)
        def _(i):
            pltpu.sync_copy(h_sh.at[pl.ds(_al(s * nb_tile + i), 200)],
                            h_out_h.at[pl.ds(_al(c * N_PAD + s * nb_tile + i),
                                             200)])

    (h_all,) = _k2(src, dst, rel, mask, nb_all, rs_all, W1)

    @functools.partial(
        pl.kernel,
        out_type=[jax.ShapeDtypeStruct((NC, R, EMB), f32)],
        mesh=_mesh(),
        compiler_params=_cparams(),
        scratch_types=[
            pltpu.VMEM_SHARED((NS, R, EMB), f32),
            pltpu.VMEM((R, EMB), f32),
            pltpu.VMEM((R, EMB), f32),
            pltpu.VMEM((R, EMB), f32),
            pltpu.VMEM((2 * L,), f32),
            pltpu.VMEM((L,), i32),
            pltpu.VMEM((L,), f32),
            pltpu.VMEM((L,), f32),
            pltpu.VMEM((L,), i32),
            pltpu.VMEM((FLUSH,), i32),
            pltpu.VMEM((FLUSH,), i32),
            pltpu.VMEM((FLUSH,), i32),
            pltpu.VMEM((FLUSH,), f32),
            pltpu.VMEM((FLUSH, EMB), f32),
            pltpu.VMEM((FLUSH, EMB), f32),
        ],
    )
    def _k3(mdst_h, mrel_h, mmsk_h, mcnt_h, h_h, rs_h, niv_h,
            hvp_h,
            hv_sh, hv_l, acc, stg, counts, idxb, ca, cbv, cvec,
            d128, d128b, r128, m128, rows0, rows1):
        c = lax.axis_index("c")
        s = lax.axis_index("s")
        w = s * NC + c
        li = _lane_iota()

        pltpu.sync_copy(niv_h, cvec)
        nv = cvec[...]

        idxb[...] = li * N + nv
        pltpu.sync_copy(rs_h.at[idxb], ca)
        idxb[...] = idxb[...] + RN_PAD
        pltpu.sync_copy(rs_h.at[idxb], cbv)
        counts[pl.ds(0, L)] = ca[...] + cbv[...]
        idxb[...] = (li * 0 + L) * N + nv
        pltpu.sync_copy(rs_h.at[idxb], ca)
        idxb[...] = idxb[...] + RN_PAD
        pltpu.sync_copy(rs_h.at[idxb], cbv)
        counts[pl.ds(L, L)] = ca[...] + cbv[...]

        @pl.loop(0, R)
        def _(r):
            hv_l[r] = jnp.zeros((EMB,), f32)

        pltpu.sync_copy(mcnt_h.at[pl.ds(_al(w * L), L)], cvec)
        cw = jnp.max(cvec[...])

        def batch_body(j, carry):
            pltpu.sync_copy(mdst_h.at[pl.ds(_al(w * MCAP + j * FLUSH), FLUSH)],
                            d128)
            pltpu.sync_copy(mrel_h.at[pl.ds(_al(w * MCAP + j * FLUSH), FLUSH)],
                            r128)
            pltpu.sync_copy(mmsk_h.at[pl.ds(_al(w * MCAP + j * FLUSH), FLUSH)],
                            m128)
            pltpu.sync_copy(h_h.at[d128], rows0)

            @pl.loop(0, FLUSH // L)
            def _(q):
                d128b[pl.ds(q * L, L)] = d128[pl.ds(q * L, L)] + N_PAD

            pltpu.sync_copy(h_h.at[d128b], rows1)

            @pl.loop(0, FLUSH // L)
            def _(q):
                r16 = r128[pl.ds(q * L, L)]
                d16 = d128[pl.ds(q * L, L)]
                cntv = plsc.load_gather(counts, [r16])
                vv = jnp.where(
                    d16 == DUMMY, 0.0,
                    _sig(m128[pl.ds(q * L, L)]) / jnp.maximum(cntv, 1.0))
                for kk in range(L):
                    jj = q * L + kk
                    sel = li == kk
                    rel_j = jnp.max(jnp.where(sel, r16, 0))
                    vj = jnp.sum(jnp.where(sel, vv, 0.0))
                    row = jnp.maximum(rows0[jj] + rows1[jj], 0.0)
                    hv_l[rel_j] = hv_l[rel_j] + vj * row

            return carry

        lax.fori_loop(0, cw // FLUSH, batch_body, 0)

        pltpu.sync_copy(hv_l, hv_sh.at[s])
        plsc.subcore_barrier()

        @pl.when(s == 0)
        def _():
            @pl.loop(0, R)
            def _(r):
                acc[r] = jnp.zeros((EMB,), f32)

            @pl.loop(0, NS)
            def _(t):
                pltpu.sync_copy(hv_sh.at[t], stg)

                @pl.loop(0, R)
                def _(r):
                    acc[r] = acc[r] + stg[r]

            pltpu.sync_copy(acc, hvp_h.at[c])

    (hvp,) = _k3(m_dst, m_rel, m_msk, m_cnt, h_all, rs_all, nidx16)

    W2p = jnp.pad(W2.astype(f32), ((0, 0), (0, 0), (0, L - CLASSES)))
    W2p = W2p.reshape(R * EMB, L)
    b2p = jnp.concatenate([b2.astype(f32), jnp.full((L - CLASSES,), -1e30, f32)])

    @functools.partial(
        pl.kernel,
        out_type=[jax.ShapeDtypeStruct((L,), f32)],
        mesh=_mesh(),
        compiler_params=_cparams(),
        scratch_types=[
            pltpu.VMEM((R, EMB), f32),
            pltpu.VMEM((R, EMB), f32),
            pltpu.VMEM((R * EMB, L), f32),
            pltpu.VMEM((L,), f32),
            pltpu.VMEM((L,), f32),
        ],
    )
    def _k4(hvp_h, w2_h, b2_h, res_h, a0, a1, w2v, bb, lgr):
        c = lax.axis_index("c")
        s = lax.axis_index("s")

        @pl.when(jnp.logical_and(c == 0, s == 0))
        def _():
            pltpu.sync_copy(hvp_h.at[0], a0)
            pltpu.sync_copy(hvp_h.at[1], a1)
            pltpu.sync_copy(w2_h, w2v)
            pltpu.sync_copy(b2_h, bb)
            lgr[...] = bb[...]
            li = _lane_iota()

            @pl.loop(0, R)
            def _(r):
                hrow = a0[r] + a1[r]
                for e in range(EMB):
                    bc = jnp.sum(jnp.where(li == e, hrow, 0.0))
                    lgr[...] = lgr[...] + bc * w2v[r * EMB + e]

            lg = lgr[...]
            mx = jnp.max(lg)
            ex = jnp.exp(lg - mx)
            sm = jnp.sum(ex)
            lgr[...] = ex / sm
            pltpu.sync_copy(lgr, res_h)

    (res,) = _k4(hvp, W2p, b2p)
    return res[:CLASSES]


def kernel(mask, W1, W2, b2, src, dst, rel, node_idx):
    return _forward(mask, W1, W2, b2, src, dst, rel, node_idx)
